# baseline (speedup 1.0000x reference)
import functools
import jax
import jax.numpy as jnp
from jax import lax
from jax.experimental import pallas as pl
from jax.experimental.pallas import tpu as pltpu, tpu_sc as plsc

SEQ = 4608
D = 768
BATCH = 4
NC = 2
NS = 16
NW = NC * NS
ROWS_W = SEQ // NW
CH = 16
NCH = ROWS_W // CH
LANE = 16
DVEC = D // LANE


def _body(x_hbm, emb_hbm, out_hbm,
          x00, x01, x02, x03, x10, x11, x12, x13, eb0, eb1,
          sx00, sx01, sx02, sx03, sx10, sx11, sx12, sx13, se0, se1,
          so00, so01, so02, so03, so10, so11, so12, so13):
    xbuf = [[x00, x01, x02, x03], [x10, x11, x12, x13]]
    ebuf = [eb0, eb1]
    sem_x = [[sx00, sx01, sx02, sx03], [sx10, sx11, sx12, sx13]]
    sem_e = [se0, se1]
    sem_o = [[so00, so01, so02, so03], [so10, so11, so12, so13]]

    wid = lax.axis_index("s") * NC + lax.axis_index("c")
    base = wid * ROWS_W

    e_desc = [None] * NCH
    x_desc = [[None] * BATCH for _ in range(NCH)]
    o_desc = [[None] * BATCH for _ in range(NCH)]

    e_desc[0] = pltpu.async_copy(
        emb_hbm.at[pl.ds(base, CH)], ebuf[0], sem_e[0])
    for b in range(BATCH):
        x_desc[0][b] = pltpu.async_copy(
            x_hbm.at[b, pl.ds(base, CH)], xbuf[0][b], sem_x[0][b])

    for c in range(NCH):
        g = c % 2
        row0 = base + c * CH
        xg = xbuf[g]
        eb = ebuf[g]

        if c + 1 < NCH:
            g2 = (c + 1) % 2
            row2 = base + (c + 1) * CH
            if c - 1 >= 0:
                for b in range(BATCH):
                    o_desc[c - 1][b].wait()
            e_desc[c + 1] = pltpu.async_copy(
                emb_hbm.at[pl.ds(row2, CH)], ebuf[g2], sem_e[g2])
            for b in range(BATCH):
                x_desc[c + 1][b] = pltpu.async_copy(
                    x_hbm.at[b, pl.ds(row2, CH)], xbuf[g2][b], sem_x[g2][b])

        e_desc[c].wait()
        for b in range(BATCH):
            x_desc[c][b].wait()

        def add_row(r, _, xg=xg, eb=eb):
            for j in range(DVEC):
                sl = pl.ds(j * LANE, LANE)
                e = eb[r, sl]
                for b in range(BATCH):
                    xg[b][r, sl] = xg[b][r, sl] + e
            return 0

        lax.fori_loop(0, CH, add_row, 0)

        for b in range(BATCH):
            o_desc[c][b] = pltpu.async_copy(
                xg[b], out_hbm.at[b, pl.ds(row0, CH)], sem_o[g][b])

    for c in range(NCH - 2, NCH):
        for b in range(BATCH):
            o_desc[c][b].wait()


def kernel(x, emb):
    mesh = plsc.VectorSubcoreMesh(core_axis_name="c", subcore_axis_name="s")
    k = functools.partial(
        pl.kernel,
        mesh=mesh,
        out_type=jax.ShapeDtypeStruct((BATCH, SEQ, D), jnp.float32),
        scratch_types=(
            [pltpu.VMEM((CH, D), jnp.float32)] * 10
            + [pltpu.SemaphoreType.DMA] * 18
        ),
    )(_body)
    return k(x, emb)
